# Initial kernel scaffold; baseline (speedup 1.0000x reference)
import jax
import jax.numpy as jnp
from jax.experimental import pallas as pl


def kernel(data, ivectors):
    raise NotImplementedError("write your pallas kernel here")


# speedup vs baseline: 1.7972x; 1.7972x over previous
import functools

import jax
import jax.numpy as jnp
from jax import lax
from jax.experimental import pallas as pl
from jax.experimental.pallas import tpu as pltpu
from jax.experimental.pallas import tpu_sc as plsc

VOCAB = 1000000
EMB = 64
B = 16384
L = 50

NUM_ROWS = B * L
NW = 32
IDX_MINOR = 128
CHUNK = 512
STREAMS = CHUNK // IDX_MINOR
BLOCKS_PER_W = NUM_ROWS // (NW * IDX_MINOR)
ITERS = BLOCKS_PER_W // STREAMS


@functools.partial(
    pl.kernel,
    out_type=jax.ShapeDtypeStruct((NUM_ROWS, EMB), jnp.float32),
    mesh=plsc.VectorSubcoreMesh(core_axis_name="c", subcore_axis_name="s"),
    scratch_types=[
        pltpu.VMEM((STREAMS, IDX_MINOR), jnp.int32),
        pltpu.VMEM((CHUNK, EMB), jnp.float32),
        pltpu.SemaphoreType.DMA,
    ],
    compiler_params=pltpu.CompilerParams(use_tc_tiling_on_sc=False),
)
def _gather_kernel(table_hbm, idx_hbm, out_hbm, idx_v, rows_v, sem):
    wid = lax.axis_index("s") * 2 + lax.axis_index("c")
    base_blk = wid * BLOCKS_PER_W

    def body(i, carry):
        blk = base_blk + i * STREAMS
        pltpu.sync_copy(idx_hbm.at[pl.ds(blk, STREAMS)], idx_v)
        cps = [
            pltpu.async_copy(
                table_hbm.at[idx_v.at[j]],
                rows_v.at[pl.ds(j * IDX_MINOR, IDX_MINOR)],
                sem,
            )
            for j in range(STREAMS)
        ]
        for c in cps:
            c.wait()
        pltpu.sync_copy(rows_v, out_hbm.at[pl.ds(blk * IDX_MINOR, CHUNK)])
        return carry

    lax.fori_loop(0, ITERS, body, 0)


def kernel(data, ivectors):
    idx = data.reshape(NUM_ROWS // IDX_MINOR, IDX_MINOR).astype(jnp.int32)
    out = _gather_kernel(ivectors, idx)
    return out.reshape(B, L, EMB)


# speedup vs baseline: 1.8742x; 1.0428x over previous
import functools

import jax
import jax.numpy as jnp
from jax import lax
from jax.experimental import pallas as pl
from jax.experimental.pallas import tpu as pltpu
from jax.experimental.pallas import tpu_sc as plsc

VOCAB = 1000000
EMB = 64
B = 16384
L = 50

NUM_ROWS = B * L
NW = 32
IDX_MINOR = 128
BLOCKS_PER_W = NUM_ROWS // (NW * IDX_MINOR)
STREAMS = 2
CHUNK = STREAMS * IDX_MINOR
NBUF = 4
CHUNKS = BLOCKS_PER_W // STREAMS
GROUPS = CHUNKS // NBUF


@functools.partial(
    pl.kernel,
    out_type=jax.ShapeDtypeStruct((NUM_ROWS, EMB), jnp.float32),
    mesh=plsc.VectorSubcoreMesh(core_axis_name="c", subcore_axis_name="s"),
    scratch_types=[
        pltpu.VMEM((BLOCKS_PER_W, IDX_MINOR), jnp.int32),
        pltpu.VMEM((NBUF, CHUNK, EMB), jnp.float32),
        pltpu.SemaphoreType.DMA((NBUF,)),
        pltpu.SemaphoreType.DMA((NBUF,)),
    ],
    compiler_params=pltpu.CompilerParams(use_tc_tiling_on_sc=False),
)
def _gather_kernel(table_hbm, idx_hbm, out_hbm, idx_v, rows_v, gsem, ssem):
    wid = lax.axis_index("s") * 2 + lax.axis_index("c")
    base_blk = wid * BLOCKS_PER_W

    pltpu.sync_copy(idx_hbm.at[pl.ds(base_blk, BLOCKS_PER_W)], idx_v)

    def fire_gather(chunk, slot):
        for j in range(STREAMS):
            pltpu.async_copy(
                table_hbm.at[idx_v.at[chunk * STREAMS + j]],
                rows_v.at[slot].at[pl.ds(j * IDX_MINOR, IDX_MINOR)],
                gsem.at[slot],
            )

    def wait_gather(slot):
        pltpu.make_async_copy(
            table_hbm.at[pl.ds(0, CHUNK)], rows_v.at[slot], gsem.at[slot]
        ).wait()

    def out_slice(chunk):
        return out_hbm.at[pl.ds((base_blk + chunk * STREAMS) * IDX_MINOR, CHUNK)]

    def wait_store(slot):
        pltpu.make_async_copy(
            rows_v.at[slot], out_hbm.at[pl.ds(0, CHUNK)], ssem.at[slot]
        ).wait()

    for b in range(NBUF):
        fire_gather(b, b)

    def group_body(g, carry):
        for b in range(NBUF):
            i = g * NBUF + b
            wait_gather(b)
            pltpu.async_copy(rows_v.at[b], out_slice(i), ssem.at[b])
            wait_store(b)

            @pl.when(g < GROUPS - 1)
            def _():
                fire_gather(i + NBUF, b)

        return carry

    lax.fori_loop(0, GROUPS, group_body, 0)


def kernel(data, ivectors):
    idx = data.reshape(NUM_ROWS // IDX_MINOR, IDX_MINOR).astype(jnp.int32)
    out = _gather_kernel(ivectors, idx)
    return out.reshape(B, L, EMB)


# speedup vs baseline: 1.8753x; 1.0006x over previous
import functools

import jax
import jax.numpy as jnp
from jax import lax
from jax.experimental import pallas as pl
from jax.experimental.pallas import tpu as pltpu
from jax.experimental.pallas import tpu_sc as plsc

VOCAB = 1000000
EMB = 64
B = 16384
L = 50

NUM_ROWS = B * L
NW = 32
IDX_MINOR = 256
BLOCKS_PER_W = NUM_ROWS // (NW * IDX_MINOR)
STREAMS = 1
CHUNK = STREAMS * IDX_MINOR
NBUF = 4
CHUNKS = BLOCKS_PER_W // STREAMS
GROUPS = CHUNKS // NBUF


@functools.partial(
    pl.kernel,
    out_type=jax.ShapeDtypeStruct((NUM_ROWS, EMB), jnp.float32),
    mesh=plsc.VectorSubcoreMesh(core_axis_name="c", subcore_axis_name="s"),
    scratch_types=[
        pltpu.VMEM((BLOCKS_PER_W, IDX_MINOR), jnp.int32),
        pltpu.VMEM((NBUF, CHUNK, EMB), jnp.float32),
        pltpu.SemaphoreType.DMA((NBUF,)),
        pltpu.SemaphoreType.DMA((NBUF,)),
    ],
    compiler_params=pltpu.CompilerParams(use_tc_tiling_on_sc=False),
)
def _gather_kernel(table_hbm, idx_hbm, out_hbm, idx_v, rows_v, gsem, ssem):
    wid = lax.axis_index("s") * 2 + lax.axis_index("c")
    base_blk = wid * BLOCKS_PER_W

    pltpu.sync_copy(idx_hbm.at[pl.ds(base_blk, BLOCKS_PER_W)], idx_v)

    def fire_gather(chunk, slot):
        for j in range(STREAMS):
            pltpu.async_copy(
                table_hbm.at[idx_v.at[chunk * STREAMS + j]],
                rows_v.at[slot].at[pl.ds(j * IDX_MINOR, IDX_MINOR)],
                gsem.at[slot],
            )

    def wait_gather(slot):
        pltpu.make_async_copy(
            table_hbm.at[pl.ds(0, CHUNK)], rows_v.at[slot], gsem.at[slot]
        ).wait()

    def out_slice(chunk):
        return out_hbm.at[pl.ds((base_blk + chunk * STREAMS) * IDX_MINOR, CHUNK)]

    def wait_store(slot):
        pltpu.make_async_copy(
            rows_v.at[slot], out_hbm.at[pl.ds(0, CHUNK)], ssem.at[slot]
        ).wait()

    for b in range(NBUF):
        fire_gather(b, b)

    def group_body(g, carry):
        for b in range(NBUF):
            i = g * NBUF + b
            wait_gather(b)
            pltpu.async_copy(rows_v.at[b], out_slice(i), ssem.at[b])
            wait_store(b)

            @pl.when(g < GROUPS - 1)
            def _():
                fire_gather(i + NBUF, b)

        return carry

    lax.fori_loop(0, GROUPS, group_body, 0)


def kernel(data, ivectors):
    idx = data.reshape(NUM_ROWS // IDX_MINOR, IDX_MINOR).astype(jnp.int32)
    out = _gather_kernel(ivectors, idx)
    return out.reshape(B, L, EMB)


# speedup vs baseline: 1.8764x; 1.0006x over previous
import functools

import jax
import jax.numpy as jnp
from jax import lax
from jax.experimental import pallas as pl
from jax.experimental.pallas import tpu as pltpu
from jax.experimental.pallas import tpu_sc as plsc

VOCAB = 1000000
HALF = VOCAB // 2
EMB = 64
B = 16384
L = 50

NUM_ROWS = B * L
NW = 32
PADL = 56

_MESH = dict(mesh=plsc.VectorSubcoreMesh(core_axis_name="c", subcore_axis_name="s"))


def _wid():
    return lax.axis_index("s") * 2 + lax.axis_index("c")


IDX_MINOR = 100
IDX_ROWS = NUM_ROWS // IDX_MINOR
ROWS_PER_W = NUM_ROWS // NW
BLOCKS_PER_W = IDX_ROWS // NW
STREAMS = 4
CHUNK = STREAMS * IDX_MINOR
BATCH_PER_CHUNK = CHUNK // L
NBUF = 2
CHUNKS = BLOCKS_PER_W // STREAMS
GROUPS = CHUNKS // NBUF


@functools.partial(
    pl.kernel,
    out_type=jax.ShapeDtypeStruct((NUM_ROWS, EMB), jnp.float32),
    scratch_types=[
        pltpu.VMEM((BLOCKS_PER_W, IDX_MINOR), jnp.int32),
        pltpu.VMEM((NBUF, CHUNK, EMB), jnp.float32),
        pltpu.SemaphoreType.DMA((NBUF,)),
        pltpu.SemaphoreType.DMA((NBUF,)),
    ],
    compiler_params=pltpu.CompilerParams(use_tc_tiling_on_sc=False),
    **_MESH,
)
def _gather_kernel(table_hbm, idx_hbm, out_hbm, idx_v, rows_v, gsem, ssem):
    wid = _wid()
    base_blk = wid * BLOCKS_PER_W
    base_b = wid * (ROWS_PER_W // L)

    pltpu.sync_copy(idx_hbm.at[pl.ds(base_blk, BLOCKS_PER_W)], idx_v)

    def fire_gather(chunk, slot):
        for j in range(STREAMS):
            pltpu.async_copy(
                table_hbm.at[idx_v.at[chunk * STREAMS + j]],
                rows_v.at[slot].at[pl.ds(j * IDX_MINOR, IDX_MINOR)],
                gsem.at[slot],
            )

    def wait_gather(slot):
        pltpu.make_async_copy(
            table_hbm.at[pl.ds(0, CHUNK)], rows_v.at[slot], gsem.at[slot]
        ).wait()

    def dummy_store(slot):
        return pltpu.make_async_copy(
            rows_v.at[slot], out_hbm.at[pl.ds(0, CHUNK)], ssem.at[slot]
        )

    for s in range(NBUF):
        fire_gather(s, s)

    def group_body(g, carry):
        for s in range(NBUF):
            i = g * NBUF + s
            wait_gather(s)
            pltpu.async_copy(
                rows_v.at[s],
                out_hbm.at[pl.ds((base_blk + i * STREAMS) * IDX_MINOR, CHUNK)],
                ssem.at[s],
            )
            dummy_store(s).wait()

            @pl.when(g < GROUPS - 1)
            def _():
                fire_gather(i + NBUF, s)

        return carry

    lax.fori_loop(0, GROUPS, group_body, 0)


def kernel(data, ivectors):
    wide = lax.optimization_barrier(ivectors.reshape(HALF, 2 * EMB))
    view = wide.reshape(VOCAB, EMB)
    idx = data.reshape(-1).astype(jnp.int32).reshape(IDX_ROWS, IDX_MINOR)
    out = _gather_kernel(view, idx)
    return out.reshape(B, L, EMB)
